# baseline (speedup 1.0000x reference)
import functools

import jax
import jax.numpy as jnp
from jax import lax
from jax.experimental import pallas as pl
from jax.experimental.pallas import tpu as pltpu
from jax.experimental.pallas import tpu_sc as plsc

N = 10000
E = 10000
NUM_GRAPHS = 8

NC = 2
NS = 16
NW = NC * NS
CHUNK = 128
NCHUNK = 3
EPT = NCHUNK * CHUNK
EPAD = NW * EPT
DC = 64
DPT = 768
NPAD = 10240
RPT = NPAD // NS
ZR = 64
ELT = EPAD // NW

ROW_BLK = 200
TC_BLK = 1000


def _mesh():
    return plsc.VectorSubcoreMesh(core_axis_name="c", subcore_axis_name="s")


_SC_PARAMS = pltpu.CompilerParams(use_tc_tiling_on_sc=False)
_SC_LG_PARAMS = pltpu.CompilerParams(use_tc_tiling_on_sc=False,
                                     needs_layout_passes=False)


def _fill_vmem(buf, rows, width, vec16):
    for r in range(rows):
        for q in range(width // 16):
            buf[r, pl.ds(q * 16, 16)] = vec16


def _deg_body(cidx_hbm, out_hbm, cidx_v, acc_v, iidx_v, deg_sh, sem):
    c = lax.axis_index("c")
    s = lax.axis_index("s")

    @pl.when(c == 0)
    def _():
        z = jnp.zeros((16,), jnp.float32)
        ones16 = jnp.ones((16,), jnp.float32)
        iota16 = lax.iota(jnp.int32, 16)
        for r in range(NPAD // 128):
            for q in range(8):
                acc_v[r, pl.ds(q * 16, 16)] = z
        for q in range(5):
            iidx_v[pl.ds(q * 16, 16)] = iota16 + q * 16
        @pl.when(s == 0)
        def _():
            pltpu.sync_copy(acc_v, deg_sh)
        base = s * DPT
        for j in range(DPT // DC):
            pltpu.sync_copy(cidx_hbm.at[pl.ds(base + j * DC, DC)],
                            cidx_v.at[j])
        for j in range(DPT // DC):
            for q in range(DC // 16):
                ci = cidx_v[j, pl.ds(q * 16, 16)]
                plsc.addupdate_scatter(
                    acc_v, [lax.shift_right_logical(ci, 7), ci & 127], ones16)
        plsc.subcore_barrier()
        pltpu.sync_copy(acc_v, deg_sh.at[iidx_v], add=True)
        plsc.subcore_barrier()
        @pl.when(s < (NPAD // 128) // 8)
        def _():
            pltpu.sync_copy(deg_sh.at[pl.ds(s * 8, 8)],
                            out_hbm.at[pl.ds(s * 8, 8)])


def _sc_degree(cidx_flat):
    return pl.kernel(
        _deg_body,
        out_type=jax.ShapeDtypeStruct((NPAD // 128, 128), jnp.float32),
        mesh=_mesh(),
        scratch_types=[
            pltpu.VMEM((DPT // DC, DC), jnp.int32),
            pltpu.VMEM((NPAD // 128, 128), jnp.float32),
            pltpu.VMEM((80,), jnp.int32),
            pltpu.VMEM_SHARED((NPAD // 128, 128), jnp.float32),
            pltpu.SemaphoreType.DMA,
        ],
        compiler_params=_SC_LG_PARAMS,
    )(cidx_flat)


def _scat_body(F, g_hbm, ridx_hbm, cidx_hbm, out_hbm,
               ridx_v, cidx_v, rows_v, zeros_v, acc_sh, sem):
    c = lax.axis_index("c")
    s = lax.axis_index("s")
    wid = s * NC + c
    base = wid * EPT
    for j in range(NCHUNK):
        pltpu.sync_copy(ridx_hbm.at[pl.ds(base + j * CHUNK, CHUNK)],
                        ridx_v.at[j])
        pltpu.sync_copy(cidx_hbm.at[pl.ds(base + j * CHUNK, CHUNK)],
                        cidx_v.at[j])
    copies = []
    for j in range(NCHUNK):
        copies.append(pltpu.async_copy(
            g_hbm.at[ridx_v.at[j]], rows_v.at[pl.ds(j * CHUNK, CHUNK)], sem))
    _fill_vmem(zeros_v, ZR, F, jnp.zeros((16,), jnp.float32))
    for zb in range(RPT // ZR):
        pltpu.sync_copy(zeros_v, acc_sh.at[pl.ds(s * RPT + zb * ZR, ZR)])
    for cp in copies:
        cp.wait()
    plsc.subcore_barrier()
    for j in range(NCHUNK):
        pltpu.sync_copy(rows_v.at[pl.ds(j * CHUNK, CHUNK)],
                        acc_sh.at[cidx_v.at[j]], add=True)
    plsc.subcore_barrier()
    pltpu.sync_copy(acc_sh.at[pl.ds(s * RPT, RPT)],
                    out_hbm.at[c, pl.ds(s * RPT, RPT)])


def _sc_scatter(g, ridx_flat, cidx_flat, F):
    return pl.kernel(
        functools.partial(_scat_body, F),
        out_type=jax.ShapeDtypeStruct((NC, NPAD, F), jnp.float32),
        mesh=_mesh(),
        scratch_types=[
            pltpu.VMEM((NCHUNK, CHUNK), jnp.int32),
            pltpu.VMEM((NCHUNK, CHUNK), jnp.int32),
            pltpu.VMEM((EPT, F), jnp.float32),
            pltpu.VMEM((ZR, F), jnp.float32),
            pltpu.VMEM_SHARED((NPAD, F), jnp.float32),
            pltpu.SemaphoreType.DMA,
        ],
        compiler_params=_SC_PARAMS,
    )(g, ridx_flat, cidx_flat)


def _eloss_body(volt_hbm, ridx_hbm, cidx_hbm, rlin_hbm, ilin_hbm, flin_hbm,
                out_hbm, volt_v, ridx_v, cidx_v, r_v, i_v, f_v, res_v, sem):
    c = lax.axis_index("c")
    s = lax.axis_index("s")
    wid = s * NC + c
    base = wid * ELT
    pltpu.sync_copy(volt_hbm, volt_v)
    pltpu.sync_copy(ridx_hbm.at[pl.ds(base, ELT)], ridx_v)
    pltpu.sync_copy(cidx_hbm.at[pl.ds(base, ELT)], cidx_v)
    pltpu.sync_copy(rlin_hbm.at[pl.ds(base, ELT)], r_v)
    pltpu.sync_copy(ilin_hbm.at[pl.ds(base, ELT)], i_v)
    pltpu.sync_copy(flin_hbm.at[pl.ds(base, ELT)], f_v)
    kvl_acc = jnp.zeros((16,), jnp.float32)
    lf_acc = jnp.zeros((16,), jnp.float32)
    for k in range(ELT // 16):
        ri = ridx_v[pl.ds(k * 16, 16)]
        ci = cidx_v[pl.ds(k * 16, 16)]
        vr = plsc.load_gather(volt_v, [ri])
        vc = plsc.load_gather(volt_v, [ci])
        vd = vr - vc
        rr = r_v[pl.ds(k * 16, 16)]
        kvl = vd - rr * i_v[pl.ds(k * 16, 16)]
        lf = vd - rr * f_v[pl.ds(k * 16, 16)]
        kvl_acc = kvl_acc + kvl * kvl
        lf_acc = lf_acc + lf * lf
    res_v[0, pl.ds(0, 16)] = kvl_acc
    res_v[1, pl.ds(0, 16)] = lf_acc
    pltpu.sync_copy(res_v, out_hbm.at[wid])


def _sc_edge_loss(volt, ridx_flat, cidx_flat, r_lin, i_lin, f_lin):
    return pl.kernel(
        _eloss_body,
        out_type=jax.ShapeDtypeStruct((NW, 2, 16), jnp.float32),
        mesh=_mesh(),
        scratch_types=[
            pltpu.VMEM((N,), jnp.float32),
            pltpu.VMEM((ELT,), jnp.int32),
            pltpu.VMEM((ELT,), jnp.int32),
            pltpu.VMEM((ELT,), jnp.float32),
            pltpu.VMEM((ELT,), jnp.float32),
            pltpu.VMEM((ELT,), jnp.float32),
            pltpu.VMEM((2, 16), jnp.float32),
            pltpu.SemaphoreType.DMA,
        ],
        compiler_params=_SC_LG_PARAMS,
    )(volt, ridx_flat, cidx_flat, r_lin, i_lin, f_lin)


def _pre_body(x_ref, w_ref, d_ref, g_ref, dinv_ref):
    deg = 1.0 + d_ref[0]
    dinv = jnp.transpose(lax.rsqrt(deg), (1, 0))
    dinv_ref[...] = dinv
    g_ref[...] = jnp.dot(x_ref[...], w_ref[...],
                         preferred_element_type=jnp.float32) * dinv


def _tc_pre(x, W1, deg):
    return pl.pallas_call(
        _pre_body,
        grid=(N // TC_BLK,),
        in_specs=[
            pl.BlockSpec((TC_BLK, 128), lambda i: (i, 0)),
            pl.BlockSpec((128, 64), lambda i: (0, 0)),
            pl.BlockSpec((1, 1, TC_BLK), lambda i: (i, 0, 0)),
        ],
        out_specs=[
            pl.BlockSpec((TC_BLK, 64), lambda i: (i, 0)),
            pl.BlockSpec((TC_BLK, 1), lambda i: (i, 0)),
        ],
        out_shape=[
            jax.ShapeDtypeStruct((N, 64), jnp.float32),
            jax.ShapeDtypeStruct((N, 1), jnp.float32),
        ],
    )(x, W1, deg)


def _layer_body(fin, fout, s0_ref, s1_ref, g_ref, dinv_ref, b_ref, w_ref,
                out_ref):
    dinv = dinv_ref[...]
    h = jax.nn.relu(dinv * (g_ref[...] + s0_ref[0] + s1_ref[0]) + b_ref[...])
    out_ref[...] = jnp.dot(h, w_ref[...],
                           preferred_element_type=jnp.float32) * dinv


def _tc_layer(scat, g, dinv, b, Wn, fin, fout):
    return pl.pallas_call(
        functools.partial(_layer_body, fin, fout),
        grid=(N // TC_BLK,),
        in_specs=[
            pl.BlockSpec((1, TC_BLK, fin), lambda i: (0, i, 0)),
            pl.BlockSpec((1, TC_BLK, fin), lambda i: (1, i, 0)),
            pl.BlockSpec((TC_BLK, fin), lambda i: (i, 0)),
            pl.BlockSpec((TC_BLK, 1), lambda i: (i, 0)),
            pl.BlockSpec((1, fin), lambda i: (0, 0)),
            pl.BlockSpec((fin, fout), lambda i: (0, 0)),
        ],
        out_specs=pl.BlockSpec((TC_BLK, fout), lambda i: (i, 0)),
        out_shape=jax.ShapeDtypeStruct((N, fout), jnp.float32),
    )(scat, scat, g, dinv, b.reshape(1, fin), Wn)


def _heads_body(s0_ref, s1_ref, g_ref, dinv_ref, b3_ref, batch_ref,
                wv1_ref, bv1_ref, wv2_ref, bv2_ref,
                wf1_ref, bf1_ref, wf2_ref, bf2_ref,
                ws1_ref, bs1_ref, ws2_ref, bs2_ref,
                volt_ref, vflat_ref, fflat_ref, dec_ref, qubo_ref, radial_ref,
                pool_acc, cnt_acc):
    i = pl.program_id(0)
    pre = dinv_ref[...] * (g_ref[...] + s0_ref[0] + s1_ref[0])
    h3 = jax.nn.relu(pre + b3_ref[...])
    hv = jax.nn.relu(jnp.dot(h3, wv1_ref[...],
                             preferred_element_type=jnp.float32) + bv1_ref[...])
    volt = jnp.dot(hv, wv2_ref[...],
                   preferred_element_type=jnp.float32) + bv2_ref[...]
    volt_ref[...] = volt
    tdims = (((0,), (1,)), ((), ()))
    vflat_ref[...] = (lax.dot_general(wv2_ref[...], hv, tdims,
                                      preferred_element_type=jnp.float32)
                      + bv2_ref[...]).reshape(1, 1, TC_BLK)
    hf = jax.nn.relu(jnp.dot(h3, wf1_ref[...],
                             preferred_element_type=jnp.float32) + bf1_ref[...])
    fflat_ref[...] = (lax.dot_general(wf2_ref[...], hf, tdims,
                                      preferred_element_type=jnp.float32)
                      + bf2_ref[...]).reshape(1, 1, TC_BLK)

    iota8 = lax.broadcasted_iota(jnp.int32, (1, NUM_GRAPHS), 1)
    onehot = (batch_ref[...] == iota8).astype(jnp.float32)
    dims = (((0,), (0,)), ((), ()))
    pool = lax.dot_general(onehot, h3, dims,
                           preferred_element_type=jnp.float32)
    ones_col = jnp.ones((TC_BLK, 1), jnp.float32)
    cnt = lax.dot_general(onehot, ones_col, dims,
                          preferred_element_type=jnp.float32)

    @pl.when(i == 0)
    def _():
        pool_acc[...] = jnp.zeros_like(pool_acc)
        cnt_acc[...] = jnp.zeros_like(cnt_acc)

    pool_acc[...] += pool
    cnt_acc[...] += cnt

    @pl.when(i == pl.num_programs(0) - 1)
    def _():
        emb = pool_acc[...] / jnp.maximum(cnt_acc[...], 1.0)
        hs = jax.nn.relu(jnp.dot(emb, ws1_ref[...],
                                 preferred_element_type=jnp.float32)
                         + bs1_ref[...])
        scores = jnp.dot(hs, ws2_ref[...],
                         preferred_element_type=jnp.float32) + bs2_ref[...]
        dec = jax.nn.sigmoid(scores)
        dec_ref[...] = dec
        qubo_ref[...] = jnp.sum(dec * dec).reshape(1, 1)
        dsum = jnp.sum(dec)
        radial_ref[...] = ((dsum - (N - 1)) ** 2 / N).reshape(1, 1)


def _tc_heads(scat, g3, dinv, b3, batch2d,
              Wv1, bv1, Wv2, bv2, Wf1, bf1, Wf2, bf2, Ws1, bs1, Ws2, bs2):
    cst = lambda i: (0, 0)
    return pl.pallas_call(
        _heads_body,
        grid=(N // TC_BLK,),
        in_specs=[
            pl.BlockSpec((1, TC_BLK, 16), lambda i: (0, i, 0)),
            pl.BlockSpec((1, TC_BLK, 16), lambda i: (1, i, 0)),
            pl.BlockSpec((TC_BLK, 16), lambda i: (i, 0)),
            pl.BlockSpec((TC_BLK, 1), lambda i: (i, 0)),
            pl.BlockSpec((1, 16), cst),
            pl.BlockSpec((TC_BLK, 1), lambda i: (i, 0)),
            pl.BlockSpec((16, 64), cst),
            pl.BlockSpec((1, 64), cst),
            pl.BlockSpec((64, 1), cst),
            pl.BlockSpec((1, 1), cst),
            pl.BlockSpec((16, 64), cst),
            pl.BlockSpec((1, 64), cst),
            pl.BlockSpec((64, 1), cst),
            pl.BlockSpec((1, 1), cst),
            pl.BlockSpec((16, 64), cst),
            pl.BlockSpec((1, 64), cst),
            pl.BlockSpec((64, 1), cst),
            pl.BlockSpec((1, 1), cst),
        ],
        out_specs=[
            pl.BlockSpec((TC_BLK, 1), lambda i: (i, 0)),
            pl.BlockSpec((1, 1, TC_BLK), lambda i: (i, 0, 0)),
            pl.BlockSpec((1, 1, TC_BLK), lambda i: (i, 0, 0)),
            pl.BlockSpec((NUM_GRAPHS, 1), cst),
            pl.BlockSpec((1, 1), cst),
            pl.BlockSpec((1, 1), cst),
        ],
        out_shape=[
            jax.ShapeDtypeStruct((N, 1), jnp.float32),
            jax.ShapeDtypeStruct((N // TC_BLK, 1, TC_BLK), jnp.float32),
            jax.ShapeDtypeStruct((N // TC_BLK, 1, TC_BLK), jnp.float32),
            jax.ShapeDtypeStruct((NUM_GRAPHS, 1), jnp.float32),
            jax.ShapeDtypeStruct((1, 1), jnp.float32),
            jax.ShapeDtypeStruct((1, 1), jnp.float32),
        ],
        scratch_shapes=[
            pltpu.VMEM((NUM_GRAPHS, 16), jnp.float32),
            pltpu.VMEM((NUM_GRAPHS, 1), jnp.float32),
        ],
    )(scat, scat, g3, dinv, b3.reshape(1, 16), batch2d,
      Wv1, bv1.reshape(1, 64), Wv2, bv2.reshape(1, 1),
      Wf1, bf1.reshape(1, 64), Wf2, bf2.reshape(1, 1),
      Ws1, bs1.reshape(1, 64), Ws2, bs2.reshape(1, 1))


def _kcl_kernel(c_ref, v_ref, inj_ref, out_ref):
    i = pl.program_id(0)
    kcl = jnp.dot(c_ref[...], v_ref[...],
                  preferred_element_type=jnp.float32) - inj_ref[...]

    @pl.when(i == 0)
    def _():
        out_ref[...] = jnp.zeros_like(out_ref)

    out_ref[...] += jnp.sum(kcl * kcl).reshape(1, 1)


def _kcl_sq_sum(C, v, inj):
    out = pl.pallas_call(
        _kcl_kernel,
        grid=(N // ROW_BLK,),
        in_specs=[
            pl.BlockSpec((ROW_BLK, N), lambda i: (i, 0)),
            pl.BlockSpec((N, 1), lambda i: (0, 0)),
            pl.BlockSpec((ROW_BLK, 1), lambda i: (i, 0)),
        ],
        out_specs=pl.BlockSpec((1, 1), lambda i: (0, 0)),
        out_shape=jax.ShapeDtypeStruct((1, 1), jnp.float32),
    )(C, v, inj.reshape(N, 1))
    return out[0, 0]


def kernel(x, edge_index, edge_attr, conductance_matrix, net_injection, line_currents, batch, W1, b1, W2, b2, W3, b3, Ws1, bs1, Ws2, bs2, Wv1, bv1, Wv2, bv2, Wf1, bf1, Wf2, bf2):
    row0, col0 = edge_index[0], edge_index[1]
    pad = EPAD - E
    zpad_i = jnp.zeros((pad,), jnp.int32)
    ridx_flat = jnp.concatenate([row0, zpad_i])
    cidx_flat = jnp.concatenate([col0, zpad_i])
    cidx_pad = jnp.concatenate(
        [col0, N + (jnp.arange(pad, dtype=jnp.int32) % (NPAD - N))])

    degp = _sc_degree(cidx_pad)
    deg3d = degp.reshape(NPAD)[:N].reshape(N // TC_BLK, 1, TC_BLK)

    g1, dinv = _tc_pre(x, W1, deg3d)
    s1 = _sc_scatter(g1, ridx_flat, cidx_pad, 64)
    g2 = _tc_layer(s1, g1, dinv, b1, W2, 64, 32)
    s2 = _sc_scatter(g2, ridx_flat, cidx_pad, 32)
    g3 = _tc_layer(s2, g2, dinv, b2, W3, 32, 16)
    s3 = _sc_scatter(g3, ridx_flat, cidx_pad, 16)

    volt2d, vflat, fflat, dec, qubo, radial = _tc_heads(
        s3, g3, dinv, b3, batch.reshape(N, 1),
        Wv1, bv1, Wv2, bv2, Wf1, bf1, Wf2, bf2, Ws1, bs1, Ws2, bs2)

    zpad_f = jnp.zeros((pad,), jnp.float32)
    r_lin = jnp.concatenate([edge_attr[:, 0], zpad_f])
    i_lin = jnp.concatenate([line_currents, zpad_f])
    f_lin = jnp.concatenate([fflat.reshape(N), zpad_f])

    eloss = _sc_edge_loss(vflat.reshape(N), ridx_flat, cidx_flat,
                          r_lin, i_lin, f_lin)
    kcl_sq = _kcl_sq_sum(conductance_matrix, volt2d, net_injection)

    kvl_sum = jnp.sum(eloss[:, 0, :])
    lf_sum = jnp.sum(eloss[:, 1, :])
    total_physics_loss = (kcl_sq / N + kvl_sum / E + lf_sum / E
                          + radial[0, 0])
    decisions = dec[:, 0]
    qubo_loss = qubo[0, 0]
    return (decisions, qubo_loss, total_physics_loss)


# speedup vs baseline: 1.2414x; 1.2414x over previous
import functools

import jax
import jax.numpy as jnp
from jax import lax
from jax.experimental import pallas as pl
from jax.experimental.pallas import tpu as pltpu
from jax.experimental.pallas import tpu_sc as plsc

N = 10000
E = 10000
NUM_GRAPHS = 8

NC = 2
NS = 16
NW = NC * NS
CHUNK = 128
NCHUNK = 3
EPT = NCHUNK * CHUNK
EPAD = NW * EPT
DC = 64
DPT = 768
NPAD = 10240
RPT = NPAD // NS
ZR = 64
ELT = EPAD // NW

ROW_BLK = 200
TC_BLK = 1000


def _mesh():
    return plsc.VectorSubcoreMesh(core_axis_name="c", subcore_axis_name="s")


_SC_PARAMS = pltpu.CompilerParams(use_tc_tiling_on_sc=False)
_SC_LG_PARAMS = pltpu.CompilerParams(use_tc_tiling_on_sc=False,
                                     needs_layout_passes=False)


def _fill_vmem(buf, rows, width, vec16):
    for r in range(rows):
        for q in range(width // 16):
            buf[r, pl.ds(q * 16, 16)] = vec16


def _deg_body(cidx_hbm, out_hbm, cidx_v, acc_v, iidx_v, deg_sh, sem):
    c = lax.axis_index("c")
    s = lax.axis_index("s")

    @pl.when(c == 0)
    def _():
        z = jnp.zeros((16,), jnp.float32)
        ones16 = jnp.ones((16,), jnp.float32)
        iota16 = lax.iota(jnp.int32, 16)
        for r in range(NPAD // 128):
            for q in range(8):
                acc_v[r, pl.ds(q * 16, 16)] = z
        for q in range(5):
            iidx_v[pl.ds(q * 16, 16)] = iota16 + q * 16
        @pl.when(s == 0)
        def _():
            pltpu.sync_copy(acc_v, deg_sh)
        base = s * DPT
        for j in range(DPT // DC):
            pltpu.sync_copy(cidx_hbm.at[pl.ds(base + j * DC, DC)],
                            cidx_v.at[j])
        for j in range(DPT // DC):
            for q in range(DC // 16):
                ci = cidx_v[j, pl.ds(q * 16, 16)]
                plsc.addupdate_scatter(
                    acc_v, [lax.shift_right_logical(ci, 7), ci & 127], ones16)
        plsc.subcore_barrier()
        pltpu.sync_copy(acc_v, deg_sh.at[iidx_v], add=True)
        plsc.subcore_barrier()
        @pl.when(s < (NPAD // 128) // 8)
        def _():
            pltpu.sync_copy(deg_sh.at[pl.ds(s * 8, 8)],
                            out_hbm.at[pl.ds(s * 8, 8)])


def _sc_degree(cidx_flat):
    return pl.kernel(
        _deg_body,
        out_type=jax.ShapeDtypeStruct((NPAD // 128, 128), jnp.float32),
        mesh=_mesh(),
        scratch_types=[
            pltpu.VMEM((DPT // DC, DC), jnp.int32),
            pltpu.VMEM((NPAD // 128, 128), jnp.float32),
            pltpu.VMEM((80,), jnp.int32),
            pltpu.VMEM_SHARED((NPAD // 128, 128), jnp.float32),
            pltpu.SemaphoreType.DMA,
        ],
        compiler_params=_SC_LG_PARAMS,
    )(cidx_flat)


def _scat_body(F, g_hbm, ridx_hbm, cidx_hbm, out_hbm,
               ridx_v, cidx_v, rows_v, zeros_v, acc_sh, sem):
    c = lax.axis_index("c")
    s = lax.axis_index("s")
    wid = s * NC + c
    pltpu.sync_copy(ridx_hbm.at[wid], ridx_v)
    pltpu.sync_copy(cidx_hbm.at[wid], cidx_v)
    copies = []
    for j in range(NCHUNK):
        copies.append(pltpu.async_copy(
            g_hbm.at[ridx_v.at[j]], rows_v.at[pl.ds(j * CHUNK, CHUNK)], sem))
    _fill_vmem(zeros_v, ZR, F, jnp.zeros((16,), jnp.float32))
    for zb in range(RPT // ZR):
        pltpu.sync_copy(zeros_v, acc_sh.at[pl.ds(s * RPT + zb * ZR, ZR)])
    for cp in copies:
        cp.wait()
    plsc.subcore_barrier()
    for j in range(NCHUNK):
        pltpu.sync_copy(rows_v.at[pl.ds(j * CHUNK, CHUNK)],
                        acc_sh.at[cidx_v.at[j]], add=True)
    plsc.subcore_barrier()
    pltpu.sync_copy(acc_sh.at[pl.ds(s * RPT, RPT)],
                    out_hbm.at[c, pl.ds(s * RPT, RPT)])


def _sc_scatter(g, ridx_flat, cidx_flat, F):
    return pl.kernel(
        functools.partial(_scat_body, F),
        out_type=jax.ShapeDtypeStruct((NC, NPAD, F), jnp.float32),
        mesh=_mesh(),
        scratch_types=[
            pltpu.VMEM((NCHUNK, CHUNK), jnp.int32),
            pltpu.VMEM((NCHUNK, CHUNK), jnp.int32),
            pltpu.VMEM((EPT, F), jnp.float32),
            pltpu.VMEM((ZR, F), jnp.float32),
            pltpu.VMEM_SHARED((NPAD, F), jnp.float32),
            pltpu.SemaphoreType.DMA,
        ],
        compiler_params=_SC_PARAMS,
    )(g, ridx_flat, cidx_flat)


def _eloss_body(volt_hbm, ridx_hbm, cidx_hbm, rlin_hbm, ilin_hbm, flin_hbm,
                out_hbm, volt_v, ridx_v, cidx_v, r_v, i_v, f_v, res_v, sem):
    c = lax.axis_index("c")
    s = lax.axis_index("s")
    wid = s * NC + c
    base = wid * ELT
    pltpu.sync_copy(volt_hbm, volt_v)
    pltpu.sync_copy(ridx_hbm.at[pl.ds(base, ELT)], ridx_v)
    pltpu.sync_copy(cidx_hbm.at[pl.ds(base, ELT)], cidx_v)
    pltpu.sync_copy(rlin_hbm.at[pl.ds(base, ELT)], r_v)
    pltpu.sync_copy(ilin_hbm.at[pl.ds(base, ELT)], i_v)
    pltpu.sync_copy(flin_hbm.at[pl.ds(base, ELT)], f_v)
    kvl_acc = jnp.zeros((16,), jnp.float32)
    lf_acc = jnp.zeros((16,), jnp.float32)
    for k in range(ELT // 16):
        ri = ridx_v[pl.ds(k * 16, 16)]
        ci = cidx_v[pl.ds(k * 16, 16)]
        vr = plsc.load_gather(volt_v, [ri])
        vc = plsc.load_gather(volt_v, [ci])
        vd = vr - vc
        rr = r_v[pl.ds(k * 16, 16)]
        kvl = vd - rr * i_v[pl.ds(k * 16, 16)]
        lf = vd - rr * f_v[pl.ds(k * 16, 16)]
        kvl_acc = kvl_acc + kvl * kvl
        lf_acc = lf_acc + lf * lf
    res_v[0, pl.ds(0, 16)] = kvl_acc
    res_v[1, pl.ds(0, 16)] = lf_acc
    pltpu.sync_copy(res_v, out_hbm.at[wid])


def _sc_edge_loss(volt, ridx_flat, cidx_flat, r_lin, i_lin, f_lin):
    return pl.kernel(
        _eloss_body,
        out_type=jax.ShapeDtypeStruct((NW, 2, 16), jnp.float32),
        mesh=_mesh(),
        scratch_types=[
            pltpu.VMEM((N,), jnp.float32),
            pltpu.VMEM((ELT,), jnp.int32),
            pltpu.VMEM((ELT,), jnp.int32),
            pltpu.VMEM((ELT,), jnp.float32),
            pltpu.VMEM((ELT,), jnp.float32),
            pltpu.VMEM((ELT,), jnp.float32),
            pltpu.VMEM((2, 16), jnp.float32),
            pltpu.SemaphoreType.DMA,
        ],
        compiler_params=_SC_LG_PARAMS,
    )(volt, ridx_flat, cidx_flat, r_lin, i_lin, f_lin)


def _pre_body(x_ref, w_ref, d_ref, g_ref, dinv_ref):
    deg = 1.0 + d_ref[0]
    dinv = jnp.transpose(lax.rsqrt(deg), (1, 0))
    dinv_ref[...] = dinv
    g_ref[...] = jnp.dot(x_ref[...], w_ref[...],
                         preferred_element_type=jnp.float32) * dinv


def _tc_pre(x, W1, deg):
    return pl.pallas_call(
        _pre_body,
        grid=(N // TC_BLK,),
        in_specs=[
            pl.BlockSpec((TC_BLK, 128), lambda i: (i, 0)),
            pl.BlockSpec((128, 64), lambda i: (0, 0)),
            pl.BlockSpec((1, 1, TC_BLK), lambda i: (i, 0, 0)),
        ],
        out_specs=[
            pl.BlockSpec((TC_BLK, 64), lambda i: (i, 0)),
            pl.BlockSpec((TC_BLK, 1), lambda i: (i, 0)),
        ],
        out_shape=[
            jax.ShapeDtypeStruct((N, 64), jnp.float32),
            jax.ShapeDtypeStruct((N, 1), jnp.float32),
        ],
    )(x, W1, deg)


def _layer_body(fin, fout, s0_ref, s1_ref, g_ref, dinv_ref, b_ref, w_ref,
                out_ref):
    dinv = dinv_ref[...]
    h = jax.nn.relu(dinv * (g_ref[...] + s0_ref[0] + s1_ref[0]) + b_ref[...])
    out_ref[...] = jnp.dot(h, w_ref[...],
                           preferred_element_type=jnp.float32) * dinv


def _tc_layer(scat, g, dinv, b, Wn, fin, fout):
    return pl.pallas_call(
        functools.partial(_layer_body, fin, fout),
        grid=(N // TC_BLK,),
        in_specs=[
            pl.BlockSpec((1, TC_BLK, fin), lambda i: (0, i, 0)),
            pl.BlockSpec((1, TC_BLK, fin), lambda i: (1, i, 0)),
            pl.BlockSpec((TC_BLK, fin), lambda i: (i, 0)),
            pl.BlockSpec((TC_BLK, 1), lambda i: (i, 0)),
            pl.BlockSpec((1, fin), lambda i: (0, 0)),
            pl.BlockSpec((fin, fout), lambda i: (0, 0)),
        ],
        out_specs=pl.BlockSpec((TC_BLK, fout), lambda i: (i, 0)),
        out_shape=jax.ShapeDtypeStruct((N, fout), jnp.float32),
    )(scat, scat, g, dinv, b.reshape(1, fin), Wn)


def _heads_body(s0_ref, s1_ref, g_ref, dinv_ref, b3_ref, batch_ref,
                wv1_ref, bv1_ref, wv2_ref, bv2_ref,
                wf1_ref, bf1_ref, wf2_ref, bf2_ref,
                ws1_ref, bs1_ref, ws2_ref, bs2_ref,
                volt_ref, vflat_ref, fflat_ref, dec_ref, qubo_ref, radial_ref,
                pool_acc, cnt_acc):
    i = pl.program_id(0)
    pre = dinv_ref[...] * (g_ref[...] + s0_ref[0] + s1_ref[0])
    h3 = jax.nn.relu(pre + b3_ref[...])
    hv = jax.nn.relu(jnp.dot(h3, wv1_ref[...],
                             preferred_element_type=jnp.float32) + bv1_ref[...])
    volt = jnp.dot(hv, wv2_ref[...],
                   preferred_element_type=jnp.float32) + bv2_ref[...]
    volt_ref[...] = volt
    tdims = (((0,), (1,)), ((), ()))
    vflat_ref[...] = (lax.dot_general(wv2_ref[...], hv, tdims,
                                      preferred_element_type=jnp.float32)
                      + bv2_ref[...]).reshape(1, 1, TC_BLK)
    hf = jax.nn.relu(jnp.dot(h3, wf1_ref[...],
                             preferred_element_type=jnp.float32) + bf1_ref[...])
    fflat_ref[...] = (lax.dot_general(wf2_ref[...], hf, tdims,
                                      preferred_element_type=jnp.float32)
                      + bf2_ref[...]).reshape(1, 1, TC_BLK)

    iota8 = lax.broadcasted_iota(jnp.int32, (1, NUM_GRAPHS), 1)
    onehot = (batch_ref[...] == iota8).astype(jnp.float32)
    dims = (((0,), (0,)), ((), ()))
    pool = lax.dot_general(onehot, h3, dims,
                           preferred_element_type=jnp.float32)
    ones_col = jnp.ones((TC_BLK, 1), jnp.float32)
    cnt = lax.dot_general(onehot, ones_col, dims,
                          preferred_element_type=jnp.float32)

    @pl.when(i == 0)
    def _():
        pool_acc[...] = jnp.zeros_like(pool_acc)
        cnt_acc[...] = jnp.zeros_like(cnt_acc)

    pool_acc[...] += pool
    cnt_acc[...] += cnt

    @pl.when(i == pl.num_programs(0) - 1)
    def _():
        emb = pool_acc[...] / jnp.maximum(cnt_acc[...], 1.0)
        hs = jax.nn.relu(jnp.dot(emb, ws1_ref[...],
                                 preferred_element_type=jnp.float32)
                         + bs1_ref[...])
        scores = jnp.dot(hs, ws2_ref[...],
                         preferred_element_type=jnp.float32) + bs2_ref[...]
        dec = jax.nn.sigmoid(scores)
        dec_ref[...] = dec
        qubo_ref[...] = jnp.sum(dec * dec).reshape(1, 1)
        dsum = jnp.sum(dec)
        radial_ref[...] = ((dsum - (N - 1)) ** 2 / N).reshape(1, 1)


def _tc_heads(scat, g3, dinv, b3, batch2d,
              Wv1, bv1, Wv2, bv2, Wf1, bf1, Wf2, bf2, Ws1, bs1, Ws2, bs2):
    cst = lambda i: (0, 0)
    return pl.pallas_call(
        _heads_body,
        grid=(N // TC_BLK,),
        in_specs=[
            pl.BlockSpec((1, TC_BLK, 16), lambda i: (0, i, 0)),
            pl.BlockSpec((1, TC_BLK, 16), lambda i: (1, i, 0)),
            pl.BlockSpec((TC_BLK, 16), lambda i: (i, 0)),
            pl.BlockSpec((TC_BLK, 1), lambda i: (i, 0)),
            pl.BlockSpec((1, 16), cst),
            pl.BlockSpec((TC_BLK, 1), lambda i: (i, 0)),
            pl.BlockSpec((16, 64), cst),
            pl.BlockSpec((1, 64), cst),
            pl.BlockSpec((64, 1), cst),
            pl.BlockSpec((1, 1), cst),
            pl.BlockSpec((16, 64), cst),
            pl.BlockSpec((1, 64), cst),
            pl.BlockSpec((64, 1), cst),
            pl.BlockSpec((1, 1), cst),
            pl.BlockSpec((16, 64), cst),
            pl.BlockSpec((1, 64), cst),
            pl.BlockSpec((64, 1), cst),
            pl.BlockSpec((1, 1), cst),
        ],
        out_specs=[
            pl.BlockSpec((TC_BLK, 1), lambda i: (i, 0)),
            pl.BlockSpec((1, 1, TC_BLK), lambda i: (i, 0, 0)),
            pl.BlockSpec((1, 1, TC_BLK), lambda i: (i, 0, 0)),
            pl.BlockSpec((NUM_GRAPHS, 1), cst),
            pl.BlockSpec((1, 1), cst),
            pl.BlockSpec((1, 1), cst),
        ],
        out_shape=[
            jax.ShapeDtypeStruct((N, 1), jnp.float32),
            jax.ShapeDtypeStruct((N // TC_BLK, 1, TC_BLK), jnp.float32),
            jax.ShapeDtypeStruct((N // TC_BLK, 1, TC_BLK), jnp.float32),
            jax.ShapeDtypeStruct((NUM_GRAPHS, 1), jnp.float32),
            jax.ShapeDtypeStruct((1, 1), jnp.float32),
            jax.ShapeDtypeStruct((1, 1), jnp.float32),
        ],
        scratch_shapes=[
            pltpu.VMEM((NUM_GRAPHS, 16), jnp.float32),
            pltpu.VMEM((NUM_GRAPHS, 1), jnp.float32),
        ],
    )(scat, scat, g3, dinv, b3.reshape(1, 16), batch2d,
      Wv1, bv1.reshape(1, 64), Wv2, bv2.reshape(1, 1),
      Wf1, bf1.reshape(1, 64), Wf2, bf2.reshape(1, 1),
      Ws1, bs1.reshape(1, 64), Ws2, bs2.reshape(1, 1))


def _kcl_kernel(c_ref, v_ref, inj_ref, out_ref):
    i = pl.program_id(0)
    kcl = jnp.dot(c_ref[...], v_ref[...],
                  preferred_element_type=jnp.float32) - inj_ref[...]

    @pl.when(i == 0)
    def _():
        out_ref[...] = jnp.zeros_like(out_ref)

    out_ref[...] += jnp.sum(kcl * kcl).reshape(1, 1)


def _kcl_sq_sum(C, v, inj):
    out = pl.pallas_call(
        _kcl_kernel,
        grid=(N // ROW_BLK,),
        in_specs=[
            pl.BlockSpec((ROW_BLK, N), lambda i: (i, 0)),
            pl.BlockSpec((N, 1), lambda i: (0, 0)),
            pl.BlockSpec((ROW_BLK, 1), lambda i: (i, 0)),
        ],
        out_specs=pl.BlockSpec((1, 1), lambda i: (0, 0)),
        out_shape=jax.ShapeDtypeStruct((1, 1), jnp.float32),
    )(C, v, inj.reshape(N, 1))
    return out[0, 0]


def kernel(x, edge_index, edge_attr, conductance_matrix, net_injection, line_currents, batch, W1, b1, W2, b2, W3, b3, Ws1, bs1, Ws2, bs2, Wv1, bv1, Wv2, bv2, Wf1, bf1, Wf2, bf2):
    row0, col0 = edge_index[0], edge_index[1]
    pad = EPAD - E
    spread = jnp.arange(pad, dtype=jnp.int32)
    ridx_flat = jnp.concatenate([row0, (spread * 37) % N])
    cidx_flat = jnp.concatenate([col0, (spread * 37) % N])
    cidx_pad = jnp.concatenate([col0, N + spread % (NPAD - N)])
    ridx3 = ridx_flat.reshape(NW, NCHUNK, CHUNK)
    cidx3 = cidx_pad.reshape(NW, NCHUNK, CHUNK)

    degp = _sc_degree(cidx_pad)
    deg3d = degp.reshape(NPAD)[:N].reshape(N // TC_BLK, 1, TC_BLK)

    g1, dinv = _tc_pre(x, W1, deg3d)
    s1 = _sc_scatter(g1, ridx3, cidx3, 64)
    g2 = _tc_layer(s1, g1, dinv, b1, W2, 64, 32)
    s2 = _sc_scatter(g2, ridx3, cidx3, 32)
    g3 = _tc_layer(s2, g2, dinv, b2, W3, 32, 16)
    s3 = _sc_scatter(g3, ridx3, cidx3, 16)

    volt2d, vflat, fflat, dec, qubo, radial = _tc_heads(
        s3, g3, dinv, b3, batch.reshape(N, 1),
        Wv1, bv1, Wv2, bv2, Wf1, bf1, Wf2, bf2, Ws1, bs1, Ws2, bs2)

    zpad_f = jnp.zeros((pad,), jnp.float32)
    r_lin = jnp.concatenate([edge_attr[:, 0], zpad_f])
    i_lin = jnp.concatenate([line_currents, zpad_f])
    f_lin = jnp.concatenate([fflat.reshape(N), zpad_f])

    eloss = _sc_edge_loss(vflat.reshape(N), ridx_flat, cidx_flat,
                          r_lin, i_lin, f_lin)
    kcl_sq = _kcl_sq_sum(conductance_matrix, volt2d, net_injection)

    kvl_sum = jnp.sum(eloss[:, 0, :])
    lf_sum = jnp.sum(eloss[:, 1, :])
    total_physics_loss = (kcl_sq / N + kvl_sum / E + lf_sum / E
                          + radial[0, 0])
    decisions = dec[:, 0]
    qubo_loss = qubo[0, 0]
    return (decisions, qubo_loss, total_physics_loss)


# speedup vs baseline: 1.3228x; 1.0656x over previous
import functools

import jax
import jax.numpy as jnp
from jax import lax
from jax.experimental import pallas as pl
from jax.experimental.pallas import tpu as pltpu
from jax.experimental.pallas import tpu_sc as plsc

N = 10000
E = 10000
NUM_GRAPHS = 8

NC = 2
NS = 16
NW = NC * NS
CHUNK = 128
NCHUNK = 3
EPT = NCHUNK * CHUNK
EPAD = NW * EPT
DC = 64
DPT = 768
NPAD = 10240
RPT = NPAD // NS
ZR = 64
ELT = EPAD // NW

ROW_BLK = 200
TC_BLK = 2000


def _mesh():
    return plsc.VectorSubcoreMesh(core_axis_name="c", subcore_axis_name="s")


_SC_PARAMS = pltpu.CompilerParams(use_tc_tiling_on_sc=False)
_SC_LG_PARAMS = pltpu.CompilerParams(use_tc_tiling_on_sc=False,
                                     needs_layout_passes=False)


def _fill_vmem(buf, rows, width, vec16):
    for r in range(rows):
        for q in range(width // 16):
            buf[r, pl.ds(q * 16, 16)] = vec16


def _deg_body(cidx_hbm, out_hbm, cidx_v, acc_v, iidx_v, deg_sh, sem):
    c = lax.axis_index("c")
    s = lax.axis_index("s")

    @pl.when(c == 0)
    def _():
        z = jnp.zeros((16,), jnp.float32)
        ones16 = jnp.ones((16,), jnp.float32)
        iota16 = lax.iota(jnp.int32, 16)
        for r in range(NPAD // 128):
            for q in range(8):
                acc_v[r, pl.ds(q * 16, 16)] = z
        for q in range(5):
            iidx_v[pl.ds(q * 16, 16)] = iota16 + q * 16
        @pl.when(s == 0)
        def _():
            pltpu.sync_copy(acc_v, deg_sh)
        base = s * DPT
        for j in range(DPT // DC):
            pltpu.sync_copy(cidx_hbm.at[pl.ds(base + j * DC, DC)],
                            cidx_v.at[j])
        for j in range(DPT // DC):
            for q in range(DC // 16):
                ci = cidx_v[j, pl.ds(q * 16, 16)]
                plsc.addupdate_scatter(
                    acc_v, [lax.shift_right_logical(ci, 7), ci & 127], ones16)
        plsc.subcore_barrier()
        pltpu.sync_copy(acc_v, deg_sh.at[iidx_v], add=True)
        plsc.subcore_barrier()
        @pl.when(s < (NPAD // 128) // 8)
        def _():
            pltpu.sync_copy(deg_sh.at[pl.ds(s * 8, 8)],
                            out_hbm.at[pl.ds(s * 8, 8)])


def _sc_degree(cidx_flat):
    return pl.kernel(
        _deg_body,
        out_type=jax.ShapeDtypeStruct((NPAD // 128, 128), jnp.float32),
        mesh=_mesh(),
        scratch_types=[
            pltpu.VMEM((DPT // DC, DC), jnp.int32),
            pltpu.VMEM((NPAD // 128, 128), jnp.float32),
            pltpu.VMEM((80,), jnp.int32),
            pltpu.VMEM_SHARED((NPAD // 128, 128), jnp.float32),
            pltpu.SemaphoreType.DMA,
        ],
        compiler_params=_SC_LG_PARAMS,
    )(cidx_flat)


def _scat_body(F, g_hbm, ridx_hbm, cidx_hbm, out_hbm,
               ridx_v, cidx_v, rows_v, zeros_v, acc_sh, sem):
    c = lax.axis_index("c")
    s = lax.axis_index("s")
    wid = s * NC + c
    pltpu.sync_copy(ridx_hbm.at[wid], ridx_v)
    pltpu.sync_copy(cidx_hbm.at[wid], cidx_v)
    copies = []
    for j in range(NCHUNK):
        copies.append(pltpu.async_copy(
            g_hbm.at[ridx_v.at[j]], rows_v.at[pl.ds(j * CHUNK, CHUNK)], sem))
    _fill_vmem(zeros_v, ZR, F, jnp.zeros((16,), jnp.float32))
    for zb in range(RPT // ZR):
        pltpu.sync_copy(zeros_v, acc_sh.at[pl.ds(s * RPT + zb * ZR, ZR)])
    for cp in copies:
        cp.wait()
    plsc.subcore_barrier()
    for j in range(NCHUNK):
        pltpu.sync_copy(rows_v.at[pl.ds(j * CHUNK, CHUNK)],
                        acc_sh.at[cidx_v.at[j]], add=True)
    plsc.subcore_barrier()
    pltpu.sync_copy(acc_sh.at[pl.ds(s * RPT, RPT)],
                    out_hbm.at[c, pl.ds(s * RPT, RPT)])


def _sc_scatter(g, ridx_flat, cidx_flat, F):
    return pl.kernel(
        functools.partial(_scat_body, F),
        out_type=jax.ShapeDtypeStruct((NC, NPAD, F), jnp.float32),
        mesh=_mesh(),
        scratch_types=[
            pltpu.VMEM((NCHUNK, CHUNK), jnp.int32),
            pltpu.VMEM((NCHUNK, CHUNK), jnp.int32),
            pltpu.VMEM((EPT, F), jnp.float32),
            pltpu.VMEM((ZR, F), jnp.float32),
            pltpu.VMEM_SHARED((NPAD, F), jnp.float32),
            pltpu.SemaphoreType.DMA,
        ],
        compiler_params=_SC_PARAMS,
    )(g, ridx_flat, cidx_flat)


def _eloss_body(volt_hbm, ridx_hbm, cidx_hbm, rlin_hbm, ilin_hbm, flin_hbm,
                out_hbm, volt_v, ridx_v, cidx_v, r_v, i_v, f_v, res_v, sem):
    c = lax.axis_index("c")
    s = lax.axis_index("s")
    wid = s * NC + c
    base = wid * ELT
    pltpu.sync_copy(volt_hbm, volt_v)
    pltpu.sync_copy(ridx_hbm.at[pl.ds(base, ELT)], ridx_v)
    pltpu.sync_copy(cidx_hbm.at[pl.ds(base, ELT)], cidx_v)
    pltpu.sync_copy(rlin_hbm.at[pl.ds(base, ELT)], r_v)
    pltpu.sync_copy(ilin_hbm.at[pl.ds(base, ELT)], i_v)
    pltpu.sync_copy(flin_hbm.at[pl.ds(base, ELT)], f_v)
    kvl_acc = jnp.zeros((16,), jnp.float32)
    lf_acc = jnp.zeros((16,), jnp.float32)
    for k in range(ELT // 16):
        ri = ridx_v[pl.ds(k * 16, 16)]
        ci = cidx_v[pl.ds(k * 16, 16)]
        vr = plsc.load_gather(volt_v, [ri])
        vc = plsc.load_gather(volt_v, [ci])
        vd = vr - vc
        rr = r_v[pl.ds(k * 16, 16)]
        kvl = vd - rr * i_v[pl.ds(k * 16, 16)]
        lf = vd - rr * f_v[pl.ds(k * 16, 16)]
        kvl_acc = kvl_acc + kvl * kvl
        lf_acc = lf_acc + lf * lf
    res_v[0, pl.ds(0, 16)] = kvl_acc
    res_v[1, pl.ds(0, 16)] = lf_acc
    pltpu.sync_copy(res_v, out_hbm.at[wid])


def _sc_edge_loss(volt, ridx_flat, cidx_flat, r_lin, i_lin, f_lin):
    return pl.kernel(
        _eloss_body,
        out_type=jax.ShapeDtypeStruct((NW, 2, 16), jnp.float32),
        mesh=_mesh(),
        scratch_types=[
            pltpu.VMEM((N,), jnp.float32),
            pltpu.VMEM((ELT,), jnp.int32),
            pltpu.VMEM((ELT,), jnp.int32),
            pltpu.VMEM((ELT,), jnp.float32),
            pltpu.VMEM((ELT,), jnp.float32),
            pltpu.VMEM((ELT,), jnp.float32),
            pltpu.VMEM((2, 16), jnp.float32),
            pltpu.SemaphoreType.DMA,
        ],
        compiler_params=_SC_LG_PARAMS,
    )(volt, ridx_flat, cidx_flat, r_lin, i_lin, f_lin)


def _pre_body(x_ref, w_ref, d_ref, g_ref, dinv_ref):
    dinv_row = lax.rsqrt(1.0 + d_ref[0])
    dinv_ref[...] = dinv_row.reshape(1, 1, TC_BLK)
    dinv = jnp.transpose(dinv_row, (1, 0))
    g_ref[...] = jnp.dot(x_ref[...], w_ref[...],
                         preferred_element_type=jnp.float32) * dinv


def _tc_pre(x, W1, deg3d):
    return pl.pallas_call(
        _pre_body,
        grid=(N // TC_BLK,),
        in_specs=[
            pl.BlockSpec((TC_BLK, 128), lambda i: (i, 0)),
            pl.BlockSpec((128, 64), lambda i: (0, 0)),
            pl.BlockSpec((1, 1, TC_BLK), lambda i: (i, 0, 0)),
        ],
        out_specs=[
            pl.BlockSpec((TC_BLK, 64), lambda i: (i, 0)),
            pl.BlockSpec((1, 1, TC_BLK), lambda i: (i, 0, 0)),
        ],
        out_shape=[
            jax.ShapeDtypeStruct((N, 64), jnp.float32),
            jax.ShapeDtypeStruct((N // TC_BLK, 1, TC_BLK), jnp.float32),
        ],
    )(x, W1, deg3d)


def _layer_body(fin, fout, s0_ref, s1_ref, g_ref, dinv_ref, b_ref, w_ref,
                out_ref):
    dinv = jnp.transpose(dinv_ref[0], (1, 0))
    s0 = s0_ref[0]
    s1 = s1_ref[0]
    h = jax.nn.relu(dinv * (g_ref[...] + s0 + s1) + b_ref[...])
    out_ref[...] = jnp.dot(h, w_ref[...],
                           preferred_element_type=jnp.float32) * dinv


def _tc_layer(scat_r, g, dinv3, b, Wn, fin, fout):
    s_specs = [pl.BlockSpec((1, TC_BLK, fin), lambda i: (0, i, 0)),
               pl.BlockSpec((1, TC_BLK, fin), lambda i: (1, i, 0))]
    s_in = scat_r
    return pl.pallas_call(
        functools.partial(_layer_body, fin, fout),
        grid=(N // TC_BLK,),
        in_specs=s_specs + [
            pl.BlockSpec((TC_BLK, fin), lambda i: (i, 0)),
            pl.BlockSpec((1, 1, TC_BLK), lambda i: (i, 0, 0)),
            pl.BlockSpec((1, fin), lambda i: (0, 0)),
            pl.BlockSpec((fin, fout), lambda i: (0, 0)),
        ],
        out_specs=pl.BlockSpec((TC_BLK, fout), lambda i: (i, 0)),
        out_shape=jax.ShapeDtypeStruct((N, fout), jnp.float32),
    )(s_in, s_in, g, dinv3, b.reshape(1, fin), Wn)


def _heads_body(s0_ref, s1_ref, g_ref, dinv_ref, b3_ref, batch_ref,
                wv1_ref, bv1_ref, wv2_ref, bv2_ref,
                wf1_ref, bf1_ref, wf2_ref, bf2_ref,
                ws1_ref, bs1_ref, ws2_ref, bs2_ref,
                volt_ref, vflat_ref, fflat_ref, dec_ref, qubo_ref, radial_ref,
                pool_acc, cnt_acc):
    i = pl.program_id(0)
    dinv = jnp.transpose(dinv_ref[0], (1, 0))
    s0 = s0_ref[0]
    s1 = s1_ref[0]
    h3 = jax.nn.relu(dinv * (g_ref[...] + s0 + s1) + b3_ref[...])
    hv = jax.nn.relu(jnp.dot(h3, wv1_ref[...],
                             preferred_element_type=jnp.float32) + bv1_ref[...])
    volt_ref[...] = jnp.dot(hv, wv2_ref[...],
                            preferred_element_type=jnp.float32) + bv2_ref[...]
    tdims = (((0,), (1,)), ((), ()))
    vflat_ref[...] = (lax.dot_general(wv2_ref[...], hv, tdims,
                                      preferred_element_type=jnp.float32)
                      + bv2_ref[...]).reshape(1, 1, TC_BLK)
    hf = jax.nn.relu(jnp.dot(h3, wf1_ref[...],
                             preferred_element_type=jnp.float32) + bf1_ref[...])
    fflat_ref[...] = (lax.dot_general(wf2_ref[...], hf, tdims,
                                      preferred_element_type=jnp.float32)
                      + bf2_ref[...]).reshape(1, 1, TC_BLK)

    iota8 = lax.broadcasted_iota(jnp.int32, (1, NUM_GRAPHS), 1)
    bcol = jnp.transpose(batch_ref[0], (1, 0))
    onehot = (bcol == iota8).astype(jnp.float32)
    dims = (((0,), (0,)), ((), ()))
    pool = lax.dot_general(onehot, h3, dims,
                           preferred_element_type=jnp.float32)
    ones_col = jnp.ones((TC_BLK, 1), jnp.float32)
    cnt = lax.dot_general(onehot, ones_col, dims,
                          preferred_element_type=jnp.float32)

    @pl.when(i == 0)
    def _():
        pool_acc[...] = jnp.zeros_like(pool_acc)
        cnt_acc[...] = jnp.zeros_like(cnt_acc)

    pool_acc[...] += pool
    cnt_acc[...] += cnt

    @pl.when(i == pl.num_programs(0) - 1)
    def _():
        emb = pool_acc[...] / jnp.maximum(cnt_acc[...], 1.0)
        hs = jax.nn.relu(jnp.dot(emb, ws1_ref[...],
                                 preferred_element_type=jnp.float32)
                         + bs1_ref[...])
        scores = jnp.dot(hs, ws2_ref[...],
                         preferred_element_type=jnp.float32) + bs2_ref[...]
        dec = jax.nn.sigmoid(scores)
        dec_ref[...] = dec
        qubo_ref[...] = jnp.sum(dec * dec).reshape(1, 1)
        dsum = jnp.sum(dec)
        radial_ref[...] = ((dsum - (N - 1)) ** 2 / N).reshape(1, 1)


def _tc_heads(scat_r, g3, dinv3, b3, batch3,
              Wv1, bv1, Wv2, bv2, Wf1, bf1, Wf2, bf2, Ws1, bs1, Ws2, bs2):
    cst = lambda i: (0, 0)
    return pl.pallas_call(
        _heads_body,
        grid=(N // TC_BLK,),
        in_specs=[
            pl.BlockSpec((1, TC_BLK, 16), lambda i: (0, i, 0)),
            pl.BlockSpec((1, TC_BLK, 16), lambda i: (1, i, 0)),
            pl.BlockSpec((TC_BLK, 16), lambda i: (i, 0)),
            pl.BlockSpec((1, 1, TC_BLK), lambda i: (i, 0, 0)),
            pl.BlockSpec((1, 16), cst),
            pl.BlockSpec((1, 1, TC_BLK), lambda i: (i, 0, 0)),
            pl.BlockSpec((16, 64), cst),
            pl.BlockSpec((1, 64), cst),
            pl.BlockSpec((64, 1), cst),
            pl.BlockSpec((1, 1), cst),
            pl.BlockSpec((16, 64), cst),
            pl.BlockSpec((1, 64), cst),
            pl.BlockSpec((64, 1), cst),
            pl.BlockSpec((1, 1), cst),
            pl.BlockSpec((16, 64), cst),
            pl.BlockSpec((1, 64), cst),
            pl.BlockSpec((64, 1), cst),
            pl.BlockSpec((1, 1), cst),
        ],
        out_specs=[
            pl.BlockSpec((TC_BLK, 1), lambda i: (i, 0)),
            pl.BlockSpec((1, 1, TC_BLK), lambda i: (i, 0, 0)),
            pl.BlockSpec((1, 1, TC_BLK), lambda i: (i, 0, 0)),
            pl.BlockSpec((NUM_GRAPHS, 1), cst),
            pl.BlockSpec((1, 1), cst),
            pl.BlockSpec((1, 1), cst),
        ],
        out_shape=[
            jax.ShapeDtypeStruct((N, 1), jnp.float32),
            jax.ShapeDtypeStruct((N // TC_BLK, 1, TC_BLK), jnp.float32),
            jax.ShapeDtypeStruct((N // TC_BLK, 1, TC_BLK), jnp.float32),
            jax.ShapeDtypeStruct((NUM_GRAPHS, 1), jnp.float32),
            jax.ShapeDtypeStruct((1, 1), jnp.float32),
            jax.ShapeDtypeStruct((1, 1), jnp.float32),
        ],
        scratch_shapes=[
            pltpu.VMEM((NUM_GRAPHS, 16), jnp.float32),
            pltpu.VMEM((NUM_GRAPHS, 1), jnp.float32),
        ],
    )(scat_r, scat_r, g3, dinv3, b3.reshape(1, 16), batch3,
      Wv1, bv1.reshape(1, 64), Wv2, bv2.reshape(1, 1),
      Wf1, bf1.reshape(1, 64), Wf2, bf2.reshape(1, 1),
      Ws1, bs1.reshape(1, 64), Ws2, bs2.reshape(1, 1))


def _kcl_kernel(c_ref, v_ref, inj_ref, out_ref):
    i = pl.program_id(0)
    kcl = jnp.dot(c_ref[...], v_ref[...],
                  preferred_element_type=jnp.float32) - inj_ref[...]

    @pl.when(i == 0)
    def _():
        out_ref[...] = jnp.zeros_like(out_ref)

    out_ref[...] += jnp.sum(kcl * kcl).reshape(1, 1)


def _kcl_sq_sum(C, v, inj):
    out = pl.pallas_call(
        _kcl_kernel,
        grid=(N // ROW_BLK,),
        in_specs=[
            pl.BlockSpec((ROW_BLK, N), lambda i: (i, 0)),
            pl.BlockSpec((N, 1), lambda i: (0, 0)),
            pl.BlockSpec((ROW_BLK, 1), lambda i: (i, 0)),
        ],
        out_specs=pl.BlockSpec((1, 1), lambda i: (0, 0)),
        out_shape=jax.ShapeDtypeStruct((1, 1), jnp.float32),
    )(C, v, inj.reshape(N, 1))
    return out[0, 0]


def kernel(x, edge_index, edge_attr, conductance_matrix, net_injection, line_currents, batch, W1, b1, W2, b2, W3, b3, Ws1, bs1, Ws2, bs2, Wv1, bv1, Wv2, bv2, Wf1, bf1, Wf2, bf2):
    row0, col0 = edge_index[0], edge_index[1]
    pad = EPAD - E
    spread = jnp.arange(pad, dtype=jnp.int32)
    ridx_flat = jnp.concatenate([row0, (spread * 37) % N])
    cidx_flat = jnp.concatenate([col0, (spread * 37) % N])
    cidx_pad = jnp.concatenate([col0, N + spread % (NPAD - N)])
    ridx3 = ridx_flat.reshape(NW, NCHUNK, CHUNK)
    cidx3 = cidx_pad.reshape(NW, NCHUNK, CHUNK)

    degp = _sc_degree(cidx_pad)
    deg3d = degp.reshape(NPAD)[:N].reshape(N // TC_BLK, 1, TC_BLK)

    g1, dinv3 = _tc_pre(x, W1, deg3d)
    s1 = _sc_scatter(g1, ridx3, cidx3, 64)
    g2 = _tc_layer(s1, g1, dinv3, b1, W2, 64, 32)
    s2 = _sc_scatter(g2, ridx3, cidx3, 32)
    g3 = _tc_layer(s2, g2, dinv3, b2, W3, 32, 16)
    s3 = _sc_scatter(g3, ridx3, cidx3, 16)

    volt2d, vflat, fflat, dec, qubo, radial = _tc_heads(
        s3, g3, dinv3, b3, batch.reshape(N // TC_BLK, 1, TC_BLK),
        Wv1, bv1, Wv2, bv2, Wf1, bf1, Wf2, bf2, Ws1, bs1, Ws2, bs2)

    zpad_f = jnp.zeros((pad,), jnp.float32)
    r_lin = jnp.concatenate([edge_attr[:, 0], zpad_f])
    i_lin = jnp.concatenate([line_currents, zpad_f])
    f_lin = jnp.concatenate([fflat.reshape(N), zpad_f])

    eloss = _sc_edge_loss(vflat.reshape(N), ridx_flat, cidx_flat,
                          r_lin, i_lin, f_lin)
    kcl_sq = _kcl_sq_sum(conductance_matrix, volt2d, net_injection)

    kvl_sum = jnp.sum(eloss[:, 0, :])
    lf_sum = jnp.sum(eloss[:, 1, :])
    total_physics_loss = (kcl_sq / N + kvl_sum / E + lf_sum / E
                          + radial[0, 0])
    decisions = dec[:, 0]
    qubo_loss = qubo[0, 0]
    return (decisions, qubo_loss, total_physics_loss)


# speedup vs baseline: 1.3270x; 1.0032x over previous
import functools

import jax
import jax.numpy as jnp
from jax import lax
from jax.experimental import pallas as pl
from jax.experimental.pallas import tpu as pltpu
from jax.experimental.pallas import tpu_sc as plsc

N = 10000
E = 10000
NUM_GRAPHS = 8

NC = 2
NS = 16
NW = NC * NS
CHUNK = 128
NCHUNK = 3
EPT = NCHUNK * CHUNK
EPAD = NW * EPT
DC = 64
DPT = 768
NPAD = 10240
RPT = NPAD // NS
ZR = 64
ELT = EPAD // NW

ROW_BLK = 400
TC_BLK = 2000


def _mesh():
    return plsc.VectorSubcoreMesh(core_axis_name="c", subcore_axis_name="s")


_SC_PARAMS = pltpu.CompilerParams(use_tc_tiling_on_sc=False)
_SC_LG_PARAMS = pltpu.CompilerParams(use_tc_tiling_on_sc=False,
                                     needs_layout_passes=False)


def _fill_vmem(buf, rows, width, vec16):
    for r in range(rows):
        for q in range(width // 16):
            buf[r, pl.ds(q * 16, 16)] = vec16


def _deg_body(cidx_hbm, out_hbm, cidx_v, acc_v, iidx_v, deg_sh, sem):
    c = lax.axis_index("c")
    s = lax.axis_index("s")

    @pl.when(c == 0)
    def _():
        z = jnp.zeros((16,), jnp.float32)
        ones16 = jnp.ones((16,), jnp.float32)
        iota16 = lax.iota(jnp.int32, 16)
        for r in range(NPAD // 128):
            for q in range(8):
                acc_v[r, pl.ds(q * 16, 16)] = z
        for q in range(5):
            iidx_v[pl.ds(q * 16, 16)] = iota16 + q * 16
        @pl.when(s == 0)
        def _():
            pltpu.sync_copy(acc_v, deg_sh)
        base = s * DPT
        for j in range(DPT // DC):
            pltpu.sync_copy(cidx_hbm.at[pl.ds(base + j * DC, DC)],
                            cidx_v.at[j])
        for j in range(DPT // DC):
            for q in range(DC // 16):
                ci = cidx_v[j, pl.ds(q * 16, 16)]
                plsc.addupdate_scatter(
                    acc_v, [lax.shift_right_logical(ci, 7), ci & 127], ones16)
        plsc.subcore_barrier()
        pltpu.sync_copy(acc_v, deg_sh.at[iidx_v], add=True)
        plsc.subcore_barrier()
        @pl.when(s < (NPAD // 128) // 8)
        def _():
            pltpu.sync_copy(deg_sh.at[pl.ds(s * 8, 8)],
                            out_hbm.at[pl.ds(s * 8, 8)])


def _sc_degree(cidx_flat):
    return pl.kernel(
        _deg_body,
        out_type=jax.ShapeDtypeStruct((NPAD // 128, 128), jnp.float32),
        mesh=_mesh(),
        scratch_types=[
            pltpu.VMEM((DPT // DC, DC), jnp.int32),
            pltpu.VMEM((NPAD // 128, 128), jnp.float32),
            pltpu.VMEM((80,), jnp.int32),
            pltpu.VMEM_SHARED((NPAD // 128, 128), jnp.float32),
            pltpu.SemaphoreType.DMA,
        ],
        compiler_params=_SC_LG_PARAMS,
    )(cidx_flat)


def _scat_body(F, g_hbm, ridx_hbm, cidx_hbm, out_hbm,
               ridx_v, cidx_v, rows_v, zeros_v, acc_sh, sem):
    c = lax.axis_index("c")
    s = lax.axis_index("s")
    wid = s * NC + c
    pltpu.sync_copy(ridx_hbm.at[wid], ridx_v)
    pltpu.sync_copy(cidx_hbm.at[wid], cidx_v)
    copies = []
    for j in range(NCHUNK):
        copies.append(pltpu.async_copy(
            g_hbm.at[ridx_v.at[j]], rows_v.at[pl.ds(j * CHUNK, CHUNK)], sem))
    _fill_vmem(zeros_v, ZR, F, jnp.zeros((16,), jnp.float32))
    for zb in range(RPT // ZR):
        pltpu.sync_copy(zeros_v, acc_sh.at[pl.ds(s * RPT + zb * ZR, ZR)])
    for cp in copies:
        cp.wait()
    plsc.subcore_barrier()
    for j in range(NCHUNK):
        pltpu.sync_copy(rows_v.at[pl.ds(j * CHUNK, CHUNK)],
                        acc_sh.at[cidx_v.at[j]], add=True)
    plsc.subcore_barrier()
    pltpu.sync_copy(acc_sh.at[pl.ds(s * RPT, RPT)],
                    out_hbm.at[c, pl.ds(s * RPT, RPT)])


def _sc_scatter(g, ridx_flat, cidx_flat, F):
    return pl.kernel(
        functools.partial(_scat_body, F),
        out_type=jax.ShapeDtypeStruct((NC, NPAD, F), jnp.float32),
        mesh=_mesh(),
        scratch_types=[
            pltpu.VMEM((NCHUNK, CHUNK), jnp.int32),
            pltpu.VMEM((NCHUNK, CHUNK), jnp.int32),
            pltpu.VMEM((EPT, F), jnp.float32),
            pltpu.VMEM((ZR, F), jnp.float32),
            pltpu.VMEM_SHARED((NPAD, F), jnp.float32),
            pltpu.SemaphoreType.DMA,
        ],
        compiler_params=_SC_PARAMS,
    )(g, ridx_flat, cidx_flat)


def _eloss_body(volt_hbm, ridx_hbm, cidx_hbm, rlin_hbm, ilin_hbm, flin_hbm,
                out_hbm, volt_v, ridx_v, cidx_v, r_v, i_v, f_v, res_v, sem):
    c = lax.axis_index("c")
    s = lax.axis_index("s")
    wid = s * NC + c
    base = wid * ELT
    pltpu.sync_copy(volt_hbm, volt_v)
    pltpu.sync_copy(ridx_hbm.at[pl.ds(base, ELT)], ridx_v)
    pltpu.sync_copy(cidx_hbm.at[pl.ds(base, ELT)], cidx_v)
    pltpu.sync_copy(rlin_hbm.at[pl.ds(base, ELT)], r_v)
    pltpu.sync_copy(ilin_hbm.at[pl.ds(base, ELT)], i_v)
    pltpu.sync_copy(flin_hbm.at[pl.ds(base, ELT)], f_v)
    kvl_acc = jnp.zeros((16,), jnp.float32)
    lf_acc = jnp.zeros((16,), jnp.float32)
    for k in range(ELT // 16):
        ri = ridx_v[pl.ds(k * 16, 16)]
        ci = cidx_v[pl.ds(k * 16, 16)]
        vr = plsc.load_gather(volt_v, [ri])
        vc = plsc.load_gather(volt_v, [ci])
        vd = vr - vc
        rr = r_v[pl.ds(k * 16, 16)]
        kvl = vd - rr * i_v[pl.ds(k * 16, 16)]
        lf = vd - rr * f_v[pl.ds(k * 16, 16)]
        kvl_acc = kvl_acc + kvl * kvl
        lf_acc = lf_acc + lf * lf
    res_v[0, pl.ds(0, 16)] = kvl_acc
    res_v[1, pl.ds(0, 16)] = lf_acc
    pltpu.sync_copy(res_v, out_hbm.at[wid])


def _sc_edge_loss(volt, ridx_flat, cidx_flat, r_lin, i_lin, f_lin):
    return pl.kernel(
        _eloss_body,
        out_type=jax.ShapeDtypeStruct((NW, 2, 16), jnp.float32),
        mesh=_mesh(),
        scratch_types=[
            pltpu.VMEM((N,), jnp.float32),
            pltpu.VMEM((ELT,), jnp.int32),
            pltpu.VMEM((ELT,), jnp.int32),
            pltpu.VMEM((ELT,), jnp.float32),
            pltpu.VMEM((ELT,), jnp.float32),
            pltpu.VMEM((ELT,), jnp.float32),
            pltpu.VMEM((2, 16), jnp.float32),
            pltpu.SemaphoreType.DMA,
        ],
        compiler_params=_SC_LG_PARAMS,
    )(volt, ridx_flat, cidx_flat, r_lin, i_lin, f_lin)


def _pre_body(x_ref, w_ref, d_ref, g_ref, dinv_ref):
    dinv_row = lax.rsqrt(1.0 + d_ref[0])
    dinv_ref[...] = dinv_row.reshape(1, 1, TC_BLK)
    dinv = jnp.transpose(dinv_row, (1, 0))
    g_ref[...] = jnp.dot(x_ref[...], w_ref[...],
                         preferred_element_type=jnp.float32) * dinv


def _tc_pre(x, W1, deg3d):
    return pl.pallas_call(
        _pre_body,
        grid=(N // TC_BLK,),
        in_specs=[
            pl.BlockSpec((TC_BLK, 128), lambda i: (i, 0)),
            pl.BlockSpec((128, 64), lambda i: (0, 0)),
            pl.BlockSpec((1, 1, TC_BLK), lambda i: (i, 0, 0)),
        ],
        out_specs=[
            pl.BlockSpec((TC_BLK, 64), lambda i: (i, 0)),
            pl.BlockSpec((1, 1, TC_BLK), lambda i: (i, 0, 0)),
        ],
        out_shape=[
            jax.ShapeDtypeStruct((N, 64), jnp.float32),
            jax.ShapeDtypeStruct((N // TC_BLK, 1, TC_BLK), jnp.float32),
        ],
    )(x, W1, deg3d)


def _layer_body(fin, fout, s0_ref, s1_ref, g_ref, dinv_ref, b_ref, w_ref,
                out_ref):
    dinv = jnp.transpose(dinv_ref[0], (1, 0))
    s0 = s0_ref[0]
    s1 = s1_ref[0]
    h = jax.nn.relu(dinv * (g_ref[...] + s0 + s1) + b_ref[...])
    out_ref[...] = jnp.dot(h, w_ref[...],
                           preferred_element_type=jnp.float32) * dinv


def _tc_layer(scat_r, g, dinv3, b, Wn, fin, fout):
    s_specs = [pl.BlockSpec((1, TC_BLK, fin), lambda i: (0, i, 0)),
               pl.BlockSpec((1, TC_BLK, fin), lambda i: (1, i, 0))]
    s_in = scat_r
    return pl.pallas_call(
        functools.partial(_layer_body, fin, fout),
        grid=(N // TC_BLK,),
        in_specs=s_specs + [
            pl.BlockSpec((TC_BLK, fin), lambda i: (i, 0)),
            pl.BlockSpec((1, 1, TC_BLK), lambda i: (i, 0, 0)),
            pl.BlockSpec((1, fin), lambda i: (0, 0)),
            pl.BlockSpec((fin, fout), lambda i: (0, 0)),
        ],
        out_specs=pl.BlockSpec((TC_BLK, fout), lambda i: (i, 0)),
        out_shape=jax.ShapeDtypeStruct((N, fout), jnp.float32),
    )(s_in, s_in, g, dinv3, b.reshape(1, fin), Wn)


def _heads_body(s0_ref, s1_ref, g_ref, dinv_ref, b3_ref, batch_ref,
                wv1_ref, bv1_ref, wv2_ref, bv2_ref,
                wf1_ref, bf1_ref, wf2_ref, bf2_ref,
                ws1_ref, bs1_ref, ws2_ref, bs2_ref,
                volt_ref, vflat_ref, fflat_ref, dec_ref, qubo_ref, radial_ref,
                pool_acc, cnt_acc):
    i = pl.program_id(0)
    dinv = jnp.transpose(dinv_ref[0], (1, 0))
    s0 = s0_ref[0]
    s1 = s1_ref[0]
    h3 = jax.nn.relu(dinv * (g_ref[...] + s0 + s1) + b3_ref[...])
    hv = jax.nn.relu(jnp.dot(h3, wv1_ref[...],
                             preferred_element_type=jnp.float32) + bv1_ref[...])
    volt_ref[...] = jnp.dot(hv, wv2_ref[...],
                            preferred_element_type=jnp.float32) + bv2_ref[...]
    tdims = (((0,), (1,)), ((), ()))
    vflat_ref[...] = (lax.dot_general(wv2_ref[...], hv, tdims,
                                      preferred_element_type=jnp.float32)
                      + bv2_ref[...]).reshape(1, 1, TC_BLK)
    hf = jax.nn.relu(jnp.dot(h3, wf1_ref[...],
                             preferred_element_type=jnp.float32) + bf1_ref[...])
    fflat_ref[...] = (lax.dot_general(wf2_ref[...], hf, tdims,
                                      preferred_element_type=jnp.float32)
                      + bf2_ref[...]).reshape(1, 1, TC_BLK)

    iota8 = lax.broadcasted_iota(jnp.int32, (1, NUM_GRAPHS), 1)
    bcol = jnp.transpose(batch_ref[0], (1, 0))
    onehot = (bcol == iota8).astype(jnp.float32)
    dims = (((0,), (0,)), ((), ()))
    pool = lax.dot_general(onehot, h3, dims,
                           preferred_element_type=jnp.float32)
    ones_col = jnp.ones((TC_BLK, 1), jnp.float32)
    cnt = lax.dot_general(onehot, ones_col, dims,
                          preferred_element_type=jnp.float32)

    @pl.when(i == 0)
    def _():
        pool_acc[...] = jnp.zeros_like(pool_acc)
        cnt_acc[...] = jnp.zeros_like(cnt_acc)

    pool_acc[...] += pool
    cnt_acc[...] += cnt

    @pl.when(i == pl.num_programs(0) - 1)
    def _():
        emb = pool_acc[...] / jnp.maximum(cnt_acc[...], 1.0)
        hs = jax.nn.relu(jnp.dot(emb, ws1_ref[...],
                                 preferred_element_type=jnp.float32)
                         + bs1_ref[...])
        scores = jnp.dot(hs, ws2_ref[...],
                         preferred_element_type=jnp.float32) + bs2_ref[...]
        dec = jax.nn.sigmoid(scores)
        dec_ref[...] = dec
        qubo_ref[...] = jnp.sum(dec * dec).reshape(1, 1)
        dsum = jnp.sum(dec)
        radial_ref[...] = ((dsum - (N - 1)) ** 2 / N).reshape(1, 1)


def _tc_heads(scat_r, g3, dinv3, b3, batch3,
              Wv1, bv1, Wv2, bv2, Wf1, bf1, Wf2, bf2, Ws1, bs1, Ws2, bs2):
    cst = lambda i: (0, 0)
    return pl.pallas_call(
        _heads_body,
        grid=(N // TC_BLK,),
        in_specs=[
            pl.BlockSpec((1, TC_BLK, 16), lambda i: (0, i, 0)),
            pl.BlockSpec((1, TC_BLK, 16), lambda i: (1, i, 0)),
            pl.BlockSpec((TC_BLK, 16), lambda i: (i, 0)),
            pl.BlockSpec((1, 1, TC_BLK), lambda i: (i, 0, 0)),
            pl.BlockSpec((1, 16), cst),
            pl.BlockSpec((1, 1, TC_BLK), lambda i: (i, 0, 0)),
            pl.BlockSpec((16, 64), cst),
            pl.BlockSpec((1, 64), cst),
            pl.BlockSpec((64, 1), cst),
            pl.BlockSpec((1, 1), cst),
            pl.BlockSpec((16, 64), cst),
            pl.BlockSpec((1, 64), cst),
            pl.BlockSpec((64, 1), cst),
            pl.BlockSpec((1, 1), cst),
            pl.BlockSpec((16, 64), cst),
            pl.BlockSpec((1, 64), cst),
            pl.BlockSpec((64, 1), cst),
            pl.BlockSpec((1, 1), cst),
        ],
        out_specs=[
            pl.BlockSpec((TC_BLK, 1), lambda i: (i, 0)),
            pl.BlockSpec((1, 1, TC_BLK), lambda i: (i, 0, 0)),
            pl.BlockSpec((1, 1, TC_BLK), lambda i: (i, 0, 0)),
            pl.BlockSpec((NUM_GRAPHS, 1), cst),
            pl.BlockSpec((1, 1), cst),
            pl.BlockSpec((1, 1), cst),
        ],
        out_shape=[
            jax.ShapeDtypeStruct((N, 1), jnp.float32),
            jax.ShapeDtypeStruct((N // TC_BLK, 1, TC_BLK), jnp.float32),
            jax.ShapeDtypeStruct((N // TC_BLK, 1, TC_BLK), jnp.float32),
            jax.ShapeDtypeStruct((NUM_GRAPHS, 1), jnp.float32),
            jax.ShapeDtypeStruct((1, 1), jnp.float32),
            jax.ShapeDtypeStruct((1, 1), jnp.float32),
        ],
        scratch_shapes=[
            pltpu.VMEM((NUM_GRAPHS, 16), jnp.float32),
            pltpu.VMEM((NUM_GRAPHS, 1), jnp.float32),
        ],
    )(scat_r, scat_r, g3, dinv3, b3.reshape(1, 16), batch3,
      Wv1, bv1.reshape(1, 64), Wv2, bv2.reshape(1, 1),
      Wf1, bf1.reshape(1, 64), Wf2, bf2.reshape(1, 1),
      Ws1, bs1.reshape(1, 64), Ws2, bs2.reshape(1, 1))


def _kcl_kernel(c_ref, v_ref, inj_ref, out_ref):
    i = pl.program_id(0)
    kcl = jnp.dot(c_ref[...], v_ref[...],
                  preferred_element_type=jnp.float32) - inj_ref[...]

    @pl.when(i == 0)
    def _():
        out_ref[...] = jnp.zeros_like(out_ref)

    out_ref[...] += jnp.sum(kcl * kcl).reshape(1, 1)


def _kcl_sq_sum(C, v, inj):
    out = pl.pallas_call(
        _kcl_kernel,
        grid=(N // ROW_BLK,),
        in_specs=[
            pl.BlockSpec((ROW_BLK, N), lambda i: (i, 0)),
            pl.BlockSpec((N, 1), lambda i: (0, 0)),
            pl.BlockSpec((ROW_BLK, 1), lambda i: (i, 0)),
        ],
        out_specs=pl.BlockSpec((1, 1), lambda i: (0, 0)),
        out_shape=jax.ShapeDtypeStruct((1, 1), jnp.float32),
    )(C, v, inj.reshape(N, 1))
    return out[0, 0]


def kernel(x, edge_index, edge_attr, conductance_matrix, net_injection, line_currents, batch, W1, b1, W2, b2, W3, b3, Ws1, bs1, Ws2, bs2, Wv1, bv1, Wv2, bv2, Wf1, bf1, Wf2, bf2):
    row0, col0 = edge_index[0], edge_index[1]
    pad = EPAD - E
    spread = jnp.arange(pad, dtype=jnp.int32)
    ridx_flat = jnp.concatenate([row0, (spread * 37) % N])
    cidx_flat = jnp.concatenate([col0, (spread * 37) % N])
    cidx_pad = jnp.concatenate([col0, N + spread % (NPAD - N)])
    ridx3 = ridx_flat.reshape(NW, NCHUNK, CHUNK)
    cidx3 = cidx_pad.reshape(NW, NCHUNK, CHUNK)

    degp = _sc_degree(cidx_pad)
    deg3d = degp.reshape(NPAD)[:N].reshape(N // TC_BLK, 1, TC_BLK)

    g1, dinv3 = _tc_pre(x, W1, deg3d)
    s1 = _sc_scatter(g1, ridx3, cidx3, 64)
    g2 = _tc_layer(s1, g1, dinv3, b1, W2, 64, 32)
    s2 = _sc_scatter(g2, ridx3, cidx3, 32)
    g3 = _tc_layer(s2, g2, dinv3, b2, W3, 32, 16)
    s3 = _sc_scatter(g3, ridx3, cidx3, 16)

    volt2d, vflat, fflat, dec, qubo, radial = _tc_heads(
        s3, g3, dinv3, b3, batch.reshape(N // TC_BLK, 1, TC_BLK),
        Wv1, bv1, Wv2, bv2, Wf1, bf1, Wf2, bf2, Ws1, bs1, Ws2, bs2)

    zpad_f = jnp.zeros((pad,), jnp.float32)
    r_lin = jnp.concatenate([edge_attr[:, 0], zpad_f])
    i_lin = jnp.concatenate([line_currents, zpad_f])
    f_lin = jnp.concatenate([fflat.reshape(N), zpad_f])

    eloss = _sc_edge_loss(vflat.reshape(N), ridx_flat, cidx_flat,
                          r_lin, i_lin, f_lin)
    kcl_sq = _kcl_sq_sum(conductance_matrix, volt2d, net_injection)

    kvl_sum = jnp.sum(eloss[:, 0, :])
    lf_sum = jnp.sum(eloss[:, 1, :])
    total_physics_loss = (kcl_sq / N + kvl_sum / E + lf_sum / E
                          + radial[0, 0])
    decisions = dec[:, 0]
    qubo_loss = qubo[0, 0]
    return (decisions, qubo_loss, total_physics_loss)
